# baseline (speedup 1.0000x reference)
import jax
import jax.numpy as jnp
from jax import lax
from jax.experimental import pallas as pl
from jax.experimental.pallas import tpu as pltpu

BM = 256
NBUF = 4


def _mm_body(a_hbm, x_vmem, o_hbm, abuf, obuf, in_sems, out_sems):
    m, kdim = a_hbm.shape
    nsteps = pl.cdiv(m, BM)

    def in_copy(i, slot):
        start = jnp.minimum(i * BM, m - BM)
        return pltpu.make_async_copy(
            a_hbm.at[pl.ds(start, BM), :], abuf.at[slot], in_sems.at[slot]
        )

    def out_copy(i, oslot):
        start = jnp.minimum(i * BM, m - BM)
        return pltpu.make_async_copy(
            obuf.at[oslot], o_hbm.at[pl.ds(start, BM), :], out_sems.at[oslot]
        )

    for i in range(NBUF - 1):
        in_copy(jnp.int32(i), i).start()

    def step(i, carry):
        slot = lax.rem(i, NBUF)
        oslot = lax.rem(i, 2)
        in_copy(i, slot).wait()

        @pl.when(i >= 2)
        def _wait_out():
            out_copy(i - 2, oslot).wait()

        obuf[oslot] = jnp.dot(
            abuf[slot],
            x_vmem[...],
            preferred_element_type=jnp.float32,
            precision=lax.Precision.DEFAULT,
        )

        nxt = i + NBUF - 1

        @pl.when(nxt < nsteps)
        def _refill():
            in_copy(nxt, lax.rem(nxt, NBUF)).start()

        out_copy(i, oslot).start()
        return carry

    lax.fori_loop(0, nsteps, step, 0)
    out_copy(nsteps - 2, lax.rem(nsteps - 2, 2)).wait()
    out_copy(nsteps - 1, lax.rem(nsteps - 1, 2)).wait()


def kernel(adj, embeds):
    m, kdim = adj.shape
    _, d = embeds.shape
    return pl.pallas_call(
        _mm_body,
        in_specs=[
            pl.BlockSpec(memory_space=pltpu.HBM),
            pl.BlockSpec(memory_space=pltpu.VMEM),
        ],
        out_specs=pl.BlockSpec(memory_space=pltpu.HBM),
        out_shape=jax.ShapeDtypeStruct((m, d), jnp.float32),
        scratch_shapes=[
            pltpu.VMEM((NBUF, BM, kdim), jnp.float32),
            pltpu.VMEM((2, BM, d), jnp.float32),
            pltpu.SemaphoreType.DMA((NBUF,)),
            pltpu.SemaphoreType.DMA((2,)),
        ],
        compiler_params=pltpu.CompilerParams(
            vmem_limit_bytes=100 * 1024 * 1024,
        ),
    )(adj, embeds)


# speedup vs baseline: 1.0352x; 1.0352x over previous
import jax
import jax.numpy as jnp
from jax import lax
from jax.experimental import pallas as pl


def _mm_kernel(a_ref, x_ref, o_ref):
    o_ref[...] = jnp.dot(
        a_ref[...],
        x_ref[...],
        preferred_element_type=jnp.float32,
        precision=lax.Precision.DEFAULT,
    )


def kernel(adj, embeds):
    m, kdim = adj.shape
    _, d = embeds.shape
    bm = 320
    return pl.pallas_call(
        _mm_kernel,
        grid=(pl.cdiv(m, bm),),
        in_specs=[
            pl.BlockSpec((bm, kdim), lambda i: (i, 0)),
            pl.BlockSpec((kdim, d), lambda i: (0, 0)),
        ],
        out_specs=pl.BlockSpec((bm, d), lambda i: (i, 0)),
        out_shape=jax.ShapeDtypeStruct((m, d), jnp.float32),
    )(adj, embeds)
